# baseline (speedup 1.0000x reference)
import functools
import jax
import jax.numpy as jnp
from jax import lax
from jax.experimental import pallas as pl
from jax.experimental.pallas import tpu as pltpu
from jax.experimental.pallas import tpu_sc as plsc

B = 1024
D = 768
P = 64
POOL = 30
L = 3
K = 3

BS1 = 32
BS = 1024
NW = 32
BW = B // NW


def _matvec_body(x_ref, w_ref, o_ref):
    x = x_ref[...].reshape(BS1 * P, D)
    w = w_ref[...]
    w_rep = jnp.concatenate([w] * BS1, axis=1)
    biota = lax.broadcasted_iota(jnp.int32, (BS1, BS1 * P), 0)
    kiota = lax.broadcasted_iota(jnp.int32, (BS1, BS1 * P), 1)
    lhs = jnp.where((kiota // P) == biota,
                    jnp.broadcast_to(w_rep, (BS1, BS1 * P)), 0.0)
    o_ref[...] = lax.dot_general(lhs, x, (((1,), (0,)), ((), ())),
                                 preferred_element_type=jnp.float32)


def _rownorm(x):
    sq = x * x
    p = sq[:, 0:128]
    for c in range(1, 6):
        p = p + sq[:, c * 128:(c + 1) * 128]
    q = p[:, 0:8]
    for t in range(1, 16):
        q = q + p[:, 8 * t:8 * t + 8]
    a0 = q[:, 0:1] + q[:, 4:5]
    a2 = q[:, 2:3] + q[:, 6:7]
    a1 = q[:, 1:2] + q[:, 5:6]
    a3 = q[:, 3:4] + q[:, 7:8]
    tot = (a0 + a2) + (a1 + a3)
    return jnp.sqrt(tot)


def _route_body(s_ref, keys_ref, b_ref, idx_ref, acc_ref):
    i = pl.program_id(0)

    @pl.when(i == 0)
    def _():
        acc_ref[0, 0] = 0.0

    s = s_ref[...] + b_ref[0, 0]
    s_norm = s / jnp.clip(_rownorm(s), 1e-12, None)
    keys = keys_ref[...]
    kn = keys / jnp.clip(_rownorm(keys), 1e-12, None)
    sim = lax.dot_general(s_norm, kn, (((1,), (1,)), ((), ())),
                          preferred_element_type=jnp.float32)

    iota = lax.broadcasted_iota(jnp.int32, (BS, POOL), 1)
    acc = 0.0
    simw = sim
    for k in range(K):
        m = jnp.max(simw, axis=1, keepdims=True)
        ismax = simw == m
        idxk = jnp.min(jnp.where(ismax, iota, POOL), axis=1, keepdims=True)
        sel = iota == idxk
        idx_ref[k:k + 1, :] = jnp.transpose(idxk, (1, 0))
        acc = acc + jnp.sum(m)
        simw = jnp.where(sel, -3e38, simw)

    acc_ref[0, 0] += acc


def _sc_gather_body(idx_hbm, pv_hbm, out_hbm, idxt, lst, buf, sem):
    wid = lax.axis_index("s") * 2 + lax.axis_index("c")
    base = wid * BW
    for k in range(K):
        pltpu.sync_copy(idx_hbm.at[k, pl.ds(base, BW)], idxt)
        for l in range(L):
            for t in range(BW // 16):
                v = idxt[pl.ds(16 * t, 16)]
                lst[pl.ds(BW * l + 16 * t, 16)] = v + POOL * l
        pltpu.async_copy(pv_hbm.at[lst], buf, sem).wait()
        for l in range(L):
            pltpu.sync_copy(buf.at[pl.ds(BW * l, BW), :],
                            out_hbm.at[L * k + l, pl.ds(base, BW), :])


def kernel(summary, prompt_keys, prompt_values, W_map, b_map):
    b2 = b_map.reshape(1, 1)
    summary_t = jnp.transpose(summary, (0, 2, 1))

    s = pl.pallas_call(
        _matvec_body,
        grid=(B // BS1,),
        in_specs=[
            pl.BlockSpec((BS1, P, D), lambda i: (i, 0, 0)),
            pl.BlockSpec((1, P), lambda i: (0, 0)),
        ],
        out_specs=pl.BlockSpec((BS1, D), lambda i: (i, 0)),
        out_shape=jax.ShapeDtypeStruct((B, D), jnp.float32),
    )(summary_t, W_map)

    idx, out_acc = pl.pallas_call(
        _route_body,
        grid=(B // BS,),
        in_specs=[
            pl.BlockSpec((BS, D), lambda i: (i, 0)),
            pl.BlockSpec((POOL, D), lambda i: (0, 0)),
            pl.BlockSpec(memory_space=pltpu.SMEM),
        ],
        out_specs=[
            pl.BlockSpec((K, BS), lambda i: (0, i)),
            pl.BlockSpec(memory_space=pltpu.SMEM),
        ],
        out_shape=[
            jax.ShapeDtypeStruct((K, B), jnp.int32),
            jax.ShapeDtypeStruct((1, 1), jnp.float32),
        ],
    )(s, prompt_keys, b2)

    pv_t = jnp.transpose(prompt_values, (1, 0, 2)).reshape(L * POOL, D)

    mesh = plsc.VectorSubcoreMesh(core_axis_name="c", subcore_axis_name="s")
    sc_gather = functools.partial(
        pl.kernel,
        mesh=mesh,
        out_type=jax.ShapeDtypeStruct((K * L, B, D), jnp.float32),
        scratch_types=[
            pltpu.VMEM((BW,), jnp.int32),
            pltpu.VMEM((L * BW,), jnp.int32),
            pltpu.VMEM((L * BW, D), jnp.float32),
            pltpu.SemaphoreType.DMA,
        ],
    )(_sc_gather_body)

    out_jmajor = sc_gather(idx, pv_t)
    batched_prompt = jnp.transpose(out_jmajor, (1, 0, 2))
    reduce_sim = out_acc[0, 0] / B
    return (batched_prompt, reduce_sim)
